# baseline (speedup 1.0000x reference)
import functools

import jax
import jax.numpy as jnp
from jax import lax
from jax.experimental import pallas as pl
from jax.experimental.pallas import tpu as pltpu
from jax.experimental.pallas import tpu_sc as plsc

NUM_FIELDS = 26
ROWS_PER_TABLE = 100001
D_MODEL = 128
BATCH = 16384

_info = plsc.get_sparse_core_info()
NC = _info.num_cores
NS = _info.num_subcores
LANES = _info.num_lanes
NW = NC * NS
BPW = BATCH // NW
NB = 64
NBLK = BPW // NB
NBUF = 8
GRP = 4
_BATCHES = [list(range(s, min(s + GRP, NUM_FIELDS)))
            for s in range(0, NUM_FIELDS, GRP)]

_mesh = plsc.VectorSubcoreMesh(core_axis_name="c", subcore_axis_name="s")


@functools.partial(
    pl.kernel,
    mesh=_mesh,
    out_type=jax.ShapeDtypeStruct((BATCH, D_MODEL), jnp.float32),
    scratch_types=[
        pltpu.VMEM((NUM_FIELDS, BPW), jnp.int32),
        pltpu.VMEM((NBUF, NB, D_MODEL), jnp.float32),
        pltpu.VMEM((NB, D_MODEL), jnp.float32),
        pltpu.SemaphoreType.DMA,
        pltpu.SemaphoreType.DMA,
        pltpu.SemaphoreType.DMA,
        pltpu.SemaphoreType.DMA,
        pltpu.SemaphoreType.DMA,
        pltpu.SemaphoreType.DMA,
        pltpu.SemaphoreType.DMA,
        pltpu.SemaphoreType.DMA,
    ],
)
def _emb_kernel(idx_hbm, tab_hbm, out_hbm, idx_v, gbuf, acc,
                s0, s1, s2, s3, s4, s5, s6, s7):
    wid = lax.axis_index("s") * NC + lax.axis_index("c")
    base = wid * BPW
    sems = (s0, s1, s2, s3, s4, s5, s6, s7)
    pltpu.sync_copy(idx_hbm.at[:, pl.ds(base, BPW)], idx_v)

    def issue(f, blk):
        pltpu.async_copy(
            tab_hbm.at[f].at[idx_v.at[f, pl.ds(blk * NB, NB)]],
            gbuf.at[f % NBUF], sems[f % NBUF])

    def wait(f, blk):
        pltpu.make_async_copy(
            tab_hbm.at[f].at[idx_v.at[f, pl.ds(blk * NB, NB)]],
            gbuf.at[f % NBUF], sems[f % NBUF]).wait()

    def accum(fields, first):
        bufs = [gbuf.at[f % NBUF] for f in fields]

        def body(r, carry):
            for c in range(D_MODEL // LANES):
                sl = pl.ds(c * LANES, LANES)
                vs = [b[r, sl] for b in bufs]
                while len(vs) > 1:
                    vs = [vs[i] + vs[i + 1] for i in range(0, len(vs) - 1, 2)] \
                        + ([vs[-1]] if len(vs) % 2 else [])
                if first:
                    acc[r, sl] = vs[0]
                else:
                    plsc.addupdate(acc.at[r, sl], vs[0])
            return carry

        lax.fori_loop(0, NB, body, 0)

    for f in _BATCHES[0] + _BATCHES[1]:
        issue(f, 0)

    def block_body(blk, carry):
        nbat = len(_BATCHES)
        for bi, fields in enumerate(_BATCHES):
            for f in fields:
                wait(f, blk)
            accum(fields, first=(bi == 0))
            if bi + 2 < nbat:
                for f in _BATCHES[bi + 2]:
                    issue(f, blk)
            elif bi == nbat - 1:
                @pl.when(blk + 1 < NBLK)
                def _():
                    for f in _BATCHES[0] + _BATCHES[1]:
                        issue(f, blk + 1)
        pltpu.sync_copy(acc, out_hbm.at[pl.ds(base + blk * NB, NB)])
        return carry

    lax.fori_loop(0, NBLK, block_body, 0)


def kernel(x, tables):
    idx_t = x.T
    return _emb_kernel(idx_t, tables)


# speedup vs baseline: 1.0537x; 1.0537x over previous
import functools

import jax
import jax.numpy as jnp
from jax import lax
from jax.experimental import pallas as pl
from jax.experimental.pallas import tpu as pltpu
from jax.experimental.pallas import tpu_sc as plsc

NUM_FIELDS = 26
ROWS_PER_TABLE = 100001
D_MODEL = 128
BATCH = 16384

_info = plsc.get_sparse_core_info()
NC = _info.num_cores
NS = _info.num_subcores
LANES = _info.num_lanes
NW = NC * NS
BPW = BATCH // NW
NB = 64
NBLK = BPW // NB
NBUF = 10
GRP = 4
_BATCHES = [list(range(s, min(s + GRP, NUM_FIELDS)))
            for s in range(0, NUM_FIELDS, GRP)]


def _slot(f):
    return f % 8 if f < 24 else 8 + (f - 24)

_mesh = plsc.VectorSubcoreMesh(core_axis_name="c", subcore_axis_name="s")


@functools.partial(
    pl.kernel,
    mesh=_mesh,
    out_type=jax.ShapeDtypeStruct((BATCH, D_MODEL), jnp.float32),
    scratch_types=[
        pltpu.VMEM((NUM_FIELDS, BPW), jnp.int32),
        pltpu.VMEM((NBUF, NB, D_MODEL), jnp.float32),
        pltpu.VMEM((NB, D_MODEL), jnp.float32),
        pltpu.SemaphoreType.DMA,
        pltpu.SemaphoreType.DMA,
        pltpu.SemaphoreType.DMA,
        pltpu.SemaphoreType.DMA,
        pltpu.SemaphoreType.DMA,
        pltpu.SemaphoreType.DMA,
        pltpu.SemaphoreType.DMA,
        pltpu.SemaphoreType.DMA,
        pltpu.SemaphoreType.DMA,
        pltpu.SemaphoreType.DMA,
    ],
)
def _emb_kernel(idx_hbm, tab_hbm, out_hbm, idx_v, gbuf, acc,
                s0, s1, s2, s3, s4, s5, s6, s7, s8, s9):
    wid = lax.axis_index("s") * NC + lax.axis_index("c")
    base = wid * BPW
    sems = (s0, s1, s2, s3, s4, s5, s6, s7, s8, s9)
    pltpu.sync_copy(idx_hbm.at[:, pl.ds(base, BPW)], idx_v)

    def issue(f, blk):
        pltpu.async_copy(
            tab_hbm.at[f].at[idx_v.at[f, pl.ds(blk * NB, NB)]],
            gbuf.at[_slot(f)], sems[_slot(f)])

    def wait(f, blk):
        pltpu.make_async_copy(
            tab_hbm.at[f].at[idx_v.at[f, pl.ds(blk * NB, NB)]],
            gbuf.at[_slot(f)], sems[_slot(f)]).wait()

    def accum(fields, first):
        bufs = [gbuf.at[_slot(f)] for f in fields]

        def body(r, carry):
            for c in range(D_MODEL // LANES):
                sl = pl.ds(c * LANES, LANES)
                vs = [b[r, sl] for b in bufs]
                while len(vs) > 1:
                    vs = [vs[i] + vs[i + 1] for i in range(0, len(vs) - 1, 2)] \
                        + ([vs[-1]] if len(vs) % 2 else [])
                if first:
                    acc[r, sl] = vs[0]
                else:
                    plsc.addupdate(acc.at[r, sl], vs[0])
            return carry

        lax.fori_loop(0, NB, body, 0)

    for f in _BATCHES[0] + _BATCHES[1]:
        issue(f, 0)

    def block_body(blk, carry):
        nbat = len(_BATCHES)
        for bi, fields in enumerate(_BATCHES):
            for f in fields:
                wait(f, blk)
            accum(fields, first=(bi == 0))
            if bi + 2 < nbat:
                for f in _BATCHES[bi + 2]:
                    issue(f, blk)
            else:
                @pl.when(blk + 1 < NBLK)
                def _():
                    for f in _BATCHES[bi + 2 - nbat]:
                        issue(f, blk + 1)
        pltpu.sync_copy(acc, out_hbm.at[pl.ds(base + blk * NB, NB)])
        return carry

    lax.fori_loop(0, NBLK, block_body, 0)


def kernel(x, tables):
    idx_t = x.T
    return _emb_kernel(idx_t, tables)


# speedup vs baseline: 1.1466x; 1.0882x over previous
import functools

import jax
import jax.numpy as jnp
from jax import lax
from jax.experimental import pallas as pl
from jax.experimental.pallas import tpu as pltpu
from jax.experimental.pallas import tpu_sc as plsc

NUM_FIELDS = 26
ROWS_PER_TABLE = 100001
D_MODEL = 128
BATCH = 16384

_info = plsc.get_sparse_core_info()
NC = _info.num_cores
NS = _info.num_subcores
LANES = _info.num_lanes
NW = NC * NS
BPW = BATCH // NW
NB = 64
NBLK = BPW // NB
NBUF = 10
GRP = 4
_BATCHES = [list(range(s, min(s + GRP, NUM_FIELDS)))
            for s in range(0, NUM_FIELDS, GRP)]


def _slot(f):
    return f % 8 if f < 24 else 8 + (f - 24)

_mesh = plsc.VectorSubcoreMesh(core_axis_name="c", subcore_axis_name="s")


@functools.partial(
    pl.kernel,
    mesh=_mesh,
    out_type=jax.ShapeDtypeStruct((BATCH, D_MODEL), jnp.float32),
    scratch_types=[
        pltpu.VMEM((NUM_FIELDS, BPW), jnp.int32),
        pltpu.VMEM((NBUF, NB, D_MODEL), jnp.float32),
        pltpu.VMEM((NB, D_MODEL), jnp.float32),
        pltpu.SemaphoreType.DMA,
        pltpu.SemaphoreType.DMA,
        pltpu.SemaphoreType.DMA,
        pltpu.SemaphoreType.DMA,
        pltpu.SemaphoreType.DMA,
        pltpu.SemaphoreType.DMA,
        pltpu.SemaphoreType.DMA,
        pltpu.SemaphoreType.DMA,
        pltpu.SemaphoreType.DMA,
        pltpu.SemaphoreType.DMA,
    ],
)
def _emb_kernel(idx_hbm, tab_hbm, out_hbm, idx_v, gbuf, acc,
                s0, s1, s2, s3, s4, s5, s6, s7, s8, s9):
    wid = lax.axis_index("s") * NC + lax.axis_index("c")
    base = wid * BPW
    sems = (s0, s1, s2, s3, s4, s5, s6, s7, s8, s9)
    pltpu.sync_copy(idx_hbm.at[:, pl.ds(base, BPW)], idx_v)

    def issue(f, blk):
        pltpu.async_copy(
            tab_hbm.at[f].at[idx_v.at[f, pl.ds(blk * NB, NB)]],
            gbuf.at[_slot(f)], sems[_slot(f)])

    def wait(f, blk):
        pltpu.make_async_copy(
            tab_hbm.at[f].at[idx_v.at[f, pl.ds(blk * NB, NB)]],
            gbuf.at[_slot(f)], sems[_slot(f)]).wait()

    def accum(fields, first):
        bufs = [gbuf.at[_slot(f)] for f in fields]

        def body(r, carry):
            for c in range(D_MODEL // LANES):
                sl = pl.ds(c * LANES, LANES)
                vs = [b[r, sl] for b in bufs]
                while len(vs) > 1:
                    vs = [vs[i] + vs[i + 1] for i in range(0, len(vs) - 1, 2)] \
                        + ([vs[-1]] if len(vs) % 2 else [])
                if first:
                    acc[r, sl] = vs[0]
                else:
                    plsc.addupdate(acc.at[r, sl], vs[0])
            return carry

        lax.fori_loop(0, NB, body, 0)

    for f in _BATCHES[0] + _BATCHES[1] + _BATCHES[-1]:
        issue(f, 0)

    def block_body(blk, carry):
        nbat = len(_BATCHES)
        for bi, fields in enumerate(_BATCHES):
            for f in fields:
                wait(f, blk)
            accum(fields, first=(bi == 0))
            if bi + 2 < nbat - 1:
                for f in _BATCHES[bi + 2]:
                    issue(f, blk)
            else:
                nxt = bi - 4 if bi < nbat - 1 else nbat - 1
                @pl.when(blk + 1 < NBLK)
                def _():
                    for f in _BATCHES[nxt]:
                        issue(f, blk + 1)
        pltpu.sync_copy(acc, out_hbm.at[pl.ds(base + blk * NB, NB)])
        return carry

    lax.fori_loop(0, NBLK, block_body, 0)


def kernel(x, tables):
    idx_t = x.T
    return _emb_kernel(idx_t, tables)
